# baseline (speedup 1.0000x reference)
import functools
import math

import jax
import jax.numpy as jnp
from jax import lax
from jax.experimental import pallas as pl
from jax.experimental.pallas import tpu as pltpu
from jax.experimental.pallas import tpu_sc as plsc

N_HEADS = 12
CAPACITY = 1024
_INT_MIN_PY = -2147483648


def _scores_body(x_ref, wr_ref, s_ref):
    xb = x_ref[...]
    Bb, TB, D = xb.shape
    s = lax.dot_general(xb.reshape(Bb * TB, D), wr_ref[...],
                        (((1,), (0,)), ((), ())),
                        precision=lax.Precision.HIGHEST,
                        preferred_element_type=jnp.float32)
    s_ref[...] = s.reshape(Bb, TB)


def _router_scores(x, Wr):
    B, T, D = x.shape
    TB = 512
    return pl.pallas_call(
        _scores_body,
        grid=(T // TB,),
        in_specs=[
            pl.BlockSpec((B, TB, D), lambda i: (0, i, 0)),
            pl.BlockSpec((D, 1), lambda i: (0, 0)),
        ],
        out_specs=pl.BlockSpec((B, TB), lambda i: (0, i)),
        out_shape=jax.ShapeDtypeStruct((B, T), jnp.float32),
    )(x, Wr)


def _make_sc_copy(N, D):
    mesh, nc, nw = _sc_mesh_info()
    rows_per_w = N // nw

    @functools.partial(
        pl.kernel, mesh=mesh,
        out_type=jax.ShapeDtypeStruct((N, D), jnp.float32),
        scratch_types=[pltpu.SemaphoreType.DMA],
    )
    def copy_k(x_hbm, out_hbm, sem):
        wid = lax.axis_index("s") * nc + lax.axis_index("c")
        base = wid * rows_per_w
        pltpu.async_copy(x_hbm.at[pl.ds(base, rows_per_w)],
                         out_hbm.at[pl.ds(base, rows_per_w)], sem).wait()

    return copy_k


def _roll(x, d, axis):
    d = d % x.shape[axis]
    if d == 0:
        return x
    if axis == 0:
        return jnp.concatenate([x[d:, :], x[:d, :]], axis=0)
    return jnp.concatenate([x[:, d:], x[:, :d]], axis=1)


def _topk_body(s_ref, idx_ref):
    s = s_ref[...]
    B, T = s.shape
    K = CAPACITY
    INT_MIN = jnp.int32(_INT_MIN_PY)
    bits = lax.bitcast_convert_type(s, jnp.int32)
    key = jnp.where(bits >= 0, bits, INT_MIN - bits)

    R, C = T // 128, 128
    LOG = (T - 1).bit_length()
    rows_io = lax.broadcasted_iota(jnp.int32, (R, C), 0)
    cols_io = lax.broadcasted_iota(jnp.int32, (R, C), 1)
    lin = rows_io * C + cols_io
    up_masks = [(lin & (1 << j)) == 0 for j in range(LOG)]
    desc_masks = [((lin >> kk) & 1) == 0 for kk in range(1, LOG)]

    out_rows = []
    for b in range(B):
        k_arr = key[b].reshape(R, C)
        g_arr = lin
        for kk in range(1, LOG + 1):
            for j in range(kk - 1, -1, -1):
                d = 1 << j
                up = up_masks[j]
                if d < C:
                    pk = jnp.where(up, _roll(k_arr, d, 1), _roll(k_arr, -d, 1))
                    pg = jnp.where(up, _roll(g_arr, d, 1), _roll(g_arr, -d, 1))
                else:
                    m = d // C
                    pk = jnp.where(up, _roll(k_arr, m, 0), _roll(k_arr, -m, 0))
                    pg = jnp.where(up, _roll(g_arr, m, 0), _roll(g_arr, -m, 0))
                beats = (k_arr > pk) | ((k_arr == pk) & (g_arr < pg))
                if kk == LOG:
                    keep = beats == up
                else:
                    keep = beats == (up == desc_masks[kk - 1])
                k_arr = jnp.where(keep, k_arr, pk)
                g_arr = jnp.where(keep, g_arr, pg)
        top = g_arr[:K // C, :].reshape(1, K)
        out_rows.append(top + b * T)
    idx_ref[...] = jnp.concatenate(out_rows, axis=0)


def _topk_global_idx(scores):
    B, T = scores.shape
    return pl.pallas_call(
        _topk_body,
        out_shape=jax.ShapeDtypeStruct((B, CAPACITY), jnp.int32),
    )(scores)


def _sc_mesh_info():
    info = plsc.get_sparse_core_info()
    return (plsc.VectorSubcoreMesh(core_axis_name="c", subcore_axis_name="s"),
            info.num_cores, info.num_cores * info.num_subcores)


def _make_sc_gather(V, D, Bn):
    mesh, nc, nw = _sc_mesh_info()
    b_per_w = Bn // nw

    @functools.partial(
        pl.kernel, mesh=mesh,
        out_type=jax.ShapeDtypeStruct((Bn, D), jnp.float32),
        scratch_types=[
            pltpu.VMEM((b_per_w,), jnp.int32),
            pltpu.VMEM((b_per_w, D), jnp.float32),
            pltpu.SemaphoreType.DMA,
        ],
    )
    def gather_k(table_hbm, idx_hbm, out_hbm, idx_v, rows_v, sem):
        wid = lax.axis_index("s") * nc + lax.axis_index("c")
        base = wid * b_per_w
        pltpu.sync_copy(idx_hbm.at[pl.ds(base, b_per_w)], idx_v)
        pltpu.async_copy(table_hbm.at[idx_v], rows_v, sem).wait()
        pltpu.sync_copy(rows_v, out_hbm.at[pl.ds(base, b_per_w)])

    return gather_k


def _make_sc_scatter(D, Bn):
    mesh, nc, nw = _sc_mesh_info()
    b_per_w = Bn // nw

    @functools.partial(
        pl.kernel, mesh=mesh,
        out_type=(),
        scratch_types=[
            pltpu.VMEM((b_per_w,), jnp.int32),
            pltpu.VMEM((b_per_w, D), jnp.float32),
            pltpu.SemaphoreType.DMA,
        ],
    )
    def scatter_k(rows_hbm, idx_hbm, out_ref, idx_v, rows_v, sem):
        wid = lax.axis_index("s") * nc + lax.axis_index("c")
        base = wid * b_per_w
        pltpu.sync_copy(idx_hbm.at[pl.ds(base, b_per_w)], idx_v)
        pltpu.sync_copy(rows_hbm.at[pl.ds(base, b_per_w)], rows_v)
        pltpu.async_copy(rows_v, out_ref.at[idx_v], sem).wait()

    return scatter_k


def _gather_rows(table, idx_flat):
    V, D = table.shape
    return _make_sc_gather(V, D, idx_flat.shape[0])(table, idx_flat)


def _scatter_rows(out_ref, rows, idx_flat):
    _make_sc_scatter(rows.shape[1], rows.shape[0])(rows, idx_flat, out_ref)


def _rms(h, g, eps=1e-6):
    norm = lax.rsqrt(jnp.mean(h * h, axis=-1, keepdims=True) + eps)
    return h * norm * g


def _attn_body(x_ref, g1_ref, wq_ref, wk_ref, wv_ref, o_ref, h1_ref):
    hpair = pl.program_id(1)

    @pl.when(hpair == 0)
    def _():
        h1_ref[...] = _rms(x_ref[...], g1_ref[...])

    h1 = h1_ref[...]
    hp = lax.Precision.DEFAULT
    q2 = lax.dot_general(h1, wq_ref[...], (((1,), (0,)), ((), ())),
                         precision=hp, preferred_element_type=jnp.float32)
    k2 = lax.dot_general(h1, wk_ref[...], (((1,), (0,)), ((), ())),
                         precision=hp, preferred_element_type=jnp.float32)
    v2 = lax.dot_general(h1, wv_ref[...], (((1,), (0,)), ((), ())),
                         precision=hp, preferred_element_type=jnp.float32)
    Tn = q2.shape[0]
    QT = 256
    scale = 1.0 / math.sqrt(64.0)
    outs = []
    for j in range(2):
        q = q2[:, j * 64:(j + 1) * 64]
        k = k2[:, j * 64:(j + 1) * 64]
        v = v2[:, j * 64:(j + 1) * 64]
        otiles = []
        for qt in range(Tn // QT):
            ext = (qt + 1) * QT
            qtile = q[qt * QT:ext, :]
            s = lax.dot_general(qtile, k[:ext, :], (((1,), (1,)), ((), ())),
                                precision=hp,
                                preferred_element_type=jnp.float32) * scale
            rio = lax.broadcasted_iota(jnp.int32, (QT, ext), 0) + qt * QT
            cio = lax.broadcasted_iota(jnp.int32, (QT, ext), 1)
            s = jnp.where(cio > rio, -1e30, s)
            mx = jnp.max(s, axis=1, keepdims=True)
            e = jnp.exp(s - mx)
            pmat = e / jnp.sum(e, axis=1, keepdims=True)
            otiles.append(lax.dot_general(pmat, v[:ext, :],
                                          (((1,), (0,)), ((), ())),
                                          precision=hp,
                                          preferred_element_type=jnp.float32))
        outs.append(jnp.concatenate(otiles, axis=0))
    o_ref[...] = jnp.concatenate(outs, axis=1)


def _attention(xs, g1, Wqkv, B, D):
    N = xs.shape[0]
    Tn = N // B
    HP = N_HEADS // 2
    return pl.pallas_call(
        _attn_body,
        grid=(B, HP),
        in_specs=[
            pl.BlockSpec((Tn, D), lambda b, h: (b, 0)),
            pl.BlockSpec((1, D), lambda b, h: (0, 0)),
            pl.BlockSpec((D, 128), lambda b, h: (0, h)),
            pl.BlockSpec((D, 128), lambda b, h: (0, h + HP)),
            pl.BlockSpec((D, 128), lambda b, h: (0, h + 2 * HP)),
        ],
        out_specs=pl.BlockSpec((Tn, 128), lambda b, h: (b, h)),
        out_shape=jax.ShapeDtypeStruct((N, D), jnp.float32),
        scratch_shapes=[pltpu.VMEM((Tn, D), jnp.float32)],
    )(xs, g1, Wqkv, Wqkv, Wqkv)


def _ffn_body(x_ref, a_ref, g2_ref, wo_ref, w1_ref, w2_ref, w3_ref, o_ref):
    xs = x_ref[...]
    y = xs + lax.dot_general(a_ref[...], wo_ref[...],
                             (((1,), (0,)), ((), ())),
                             preferred_element_type=jnp.float32)
    h2 = _rms(y, g2_ref[...])
    a = lax.dot_general(h2, w1_ref[...], (((1,), (0,)), ((), ())),
                        preferred_element_type=jnp.float32)
    b = lax.dot_general(h2, w2_ref[...], (((1,), (0,)), ((), ())),
                        preferred_element_type=jnp.float32)
    act = (a / (1.0 + jnp.exp(-a))) * b
    ff = lax.dot_general(act, w3_ref[...], (((1,), (0,)), ((), ())),
                         preferred_element_type=jnp.float32)
    o_ref[...] = y + ff


def _out_ffn(xs, attn, g2, Wo, W1, W2, W3):
    N, D = xs.shape
    F = W1.shape[1]
    TM = 512
    return pl.pallas_call(
        _ffn_body,
        grid=(N // TM,),
        in_specs=[
            pl.BlockSpec((TM, D), lambda i: (i, 0)),
            pl.BlockSpec((TM, D), lambda i: (i, 0)),
            pl.BlockSpec((1, D), lambda i: (0, 0)),
            pl.BlockSpec((D, D), lambda i: (0, 0)),
            pl.BlockSpec((D, F), lambda i: (0, 0)),
            pl.BlockSpec((D, F), lambda i: (0, 0)),
            pl.BlockSpec((F, D), lambda i: (0, 0)),
        ],
        out_specs=pl.BlockSpec((TM, D), lambda i: (i, 0)),
        out_shape=jax.ShapeDtypeStruct((N, D), jnp.float32),
    )(xs, attn, g2, Wo, W1, W2, W3)


def kernel(x, Wr, g1, g2, Wqkv, Wo, W1, W2, W3):
    B, T, D = x.shape
    x2d = x.reshape(B * T, D)
    out = _make_sc_copy(B * T, D)(x2d)
    scores = _router_scores(x, Wr)
    idx_g = _topk_global_idx(scores)
    idx_flat = idx_g.reshape(B * CAPACITY)
    xs = _gather_rows(x2d, idx_flat)
    attn = _attention(xs, g1.reshape(1, D), Wqkv, B, D)
    xproc = _out_ffn(xs, attn, g2.reshape(1, D), Wo, W1, W2, W3)
    out_ref = jax.new_ref(out)
    _scatter_rows(out_ref, xproc, idx_flat)
    return jax.freeze(out_ref).reshape(B, T, D)


# speedup vs baseline: 8.9592x; 8.9592x over previous
import functools
import math

import jax
import jax.numpy as jnp
from jax import lax
from jax.experimental import pallas as pl
from jax.experimental.pallas import tpu as pltpu
from jax.experimental.pallas import tpu_sc as plsc

N_HEADS = 12
CAPACITY = 1024
_INT_MIN_PY = -2147483648


def _scores_body(x_ref, wr_ref, s_ref):
    xb = x_ref[...]
    Bb, TB, D = xb.shape
    s = lax.dot_general(xb.reshape(Bb * TB, D), wr_ref[...],
                        (((1,), (0,)), ((), ())),
                        precision=lax.Precision.HIGHEST,
                        preferred_element_type=jnp.float32)
    s_ref[...] = s.reshape(Bb, TB)


def _router_scores(x, Wr):
    B, T, D = x.shape
    TB = 512
    return pl.pallas_call(
        _scores_body,
        grid=(T // TB,),
        in_specs=[
            pl.BlockSpec((B, TB, D), lambda i: (0, i, 0)),
            pl.BlockSpec((D, 1), lambda i: (0, 0)),
        ],
        out_specs=pl.BlockSpec((B, TB), lambda i: (0, i)),
        out_shape=jax.ShapeDtypeStruct((B, T), jnp.float32),
    )(x, Wr)


def _make_sc_copy(N, D):
    mesh, nc, nw = _sc_mesh_info()
    rows_per_w = N // nw
    CHUNK = 64
    nch = rows_per_w // CHUNK

    @functools.partial(
        pl.kernel, mesh=mesh,
        out_type=jax.ShapeDtypeStruct((N, D), jnp.float32),
        scratch_types=[
            pltpu.VMEM((CHUNK, D), jnp.float32),
            pltpu.VMEM((CHUNK, D), jnp.float32),
            pltpu.SemaphoreType.DMA,
            pltpu.SemaphoreType.DMA,
            pltpu.SemaphoreType.DMA,
            pltpu.SemaphoreType.DMA,
        ],
    )
    def copy_k(x_hbm, out_hbm, buf0, buf1, in0, in1, out0, out1):
        wid = lax.axis_index("s") * nc + lax.axis_index("c")
        base = wid * rows_per_w
        bufs = (buf0, buf1)
        insems = (in0, in1)
        outsems = (out0, out1)
        for c in range(nch):
            sl = c % 2
            r0 = base + c * CHUNK
            if c >= 2:
                pltpu.make_async_copy(bufs[sl], out_hbm.at[pl.ds(r0, CHUNK)],
                                      outsems[sl]).wait()
            cp_in = pltpu.make_async_copy(x_hbm.at[pl.ds(r0, CHUNK)],
                                          bufs[sl], insems[sl])
            cp_in.start()
            cp_in.wait()
            pltpu.make_async_copy(bufs[sl], out_hbm.at[pl.ds(r0, CHUNK)],
                                  outsems[sl]).start()
        for sl in range(2):
            c = nch - 2 + sl
            r0 = base + c * CHUNK
            pltpu.make_async_copy(bufs[c % 2], out_hbm.at[pl.ds(r0, CHUNK)],
                                  outsems[c % 2]).wait()

    return copy_k


def _roll(x, d, axis):
    d = d % x.shape[axis]
    if d == 0:
        return x
    if axis == 0:
        return jnp.concatenate([x[d:, :], x[:d, :]], axis=0)
    return jnp.concatenate([x[:, d:], x[:, :d]], axis=1)


def _topk_body(s_ref, idx_ref):
    s = s_ref[...]
    B, T = s.shape
    K = CAPACITY
    INT_MIN = jnp.int32(_INT_MIN_PY)
    bits = lax.bitcast_convert_type(s, jnp.int32)
    key = jnp.where(bits >= 0, bits, INT_MIN - bits)

    R, C = T // 128, 128
    LOG = (T - 1).bit_length()
    rows_io = lax.broadcasted_iota(jnp.int32, (R, C), 0)
    cols_io = lax.broadcasted_iota(jnp.int32, (R, C), 1)
    lin = rows_io * C + cols_io
    up_masks = [(lin & (1 << j)) == 0 for j in range(LOG)]
    desc_masks = [((lin >> kk) & 1) == 0 for kk in range(1, LOG)]

    out_rows = []
    for b in range(B):
        k_arr = key[b].reshape(R, C)
        g_arr = lin
        for kk in range(1, LOG + 1):
            for j in range(kk - 1, -1, -1):
                d = 1 << j
                up = up_masks[j]
                if d < C:
                    pk = jnp.where(up, _roll(k_arr, d, 1), _roll(k_arr, -d, 1))
                    pg = jnp.where(up, _roll(g_arr, d, 1), _roll(g_arr, -d, 1))
                else:
                    m = d // C
                    pk = jnp.where(up, _roll(k_arr, m, 0), _roll(k_arr, -m, 0))
                    pg = jnp.where(up, _roll(g_arr, m, 0), _roll(g_arr, -m, 0))
                beats = (k_arr > pk) | ((k_arr == pk) & (g_arr < pg))
                if kk == LOG:
                    keep = beats == up
                else:
                    keep = beats == (up == desc_masks[kk - 1])
                k_arr = jnp.where(keep, k_arr, pk)
                g_arr = jnp.where(keep, g_arr, pg)
        top = g_arr[:K // C, :].reshape(1, K)
        out_rows.append(top + b * T)
    idx_ref[...] = jnp.concatenate(out_rows, axis=0)


def _topk_global_idx(scores):
    B, T = scores.shape
    return pl.pallas_call(
        _topk_body,
        out_shape=jax.ShapeDtypeStruct((B, CAPACITY), jnp.int32),
    )(scores)


def _sc_mesh_info():
    info = plsc.get_sparse_core_info()
    return (plsc.VectorSubcoreMesh(core_axis_name="c", subcore_axis_name="s"),
            info.num_cores, info.num_cores * info.num_subcores)


def _make_sc_gather(V, D, Bn):
    mesh, nc, nw = _sc_mesh_info()
    b_per_w = Bn // nw

    @functools.partial(
        pl.kernel, mesh=mesh,
        out_type=jax.ShapeDtypeStruct((Bn, D), jnp.float32),
        scratch_types=[
            pltpu.VMEM((b_per_w,), jnp.int32),
            pltpu.VMEM((b_per_w, D), jnp.float32),
            pltpu.SemaphoreType.DMA,
        ],
    )
    def gather_k(table_hbm, idx_hbm, out_hbm, idx_v, rows_v, sem):
        wid = lax.axis_index("s") * nc + lax.axis_index("c")
        base = wid * b_per_w
        pltpu.sync_copy(idx_hbm.at[pl.ds(base, b_per_w)], idx_v)
        pltpu.async_copy(table_hbm.at[idx_v], rows_v, sem).wait()
        pltpu.sync_copy(rows_v, out_hbm.at[pl.ds(base, b_per_w)])

    return gather_k


def _make_sc_scatter(D, Bn):
    mesh, nc, nw = _sc_mesh_info()
    b_per_w = Bn // nw

    @functools.partial(
        pl.kernel, mesh=mesh,
        out_type=(),
        scratch_types=[
            pltpu.VMEM((b_per_w,), jnp.int32),
            pltpu.VMEM((b_per_w, D), jnp.float32),
            pltpu.SemaphoreType.DMA,
        ],
    )
    def scatter_k(rows_hbm, idx_hbm, out_ref, idx_v, rows_v, sem):
        wid = lax.axis_index("s") * nc + lax.axis_index("c")
        base = wid * b_per_w
        pltpu.sync_copy(idx_hbm.at[pl.ds(base, b_per_w)], idx_v)
        pltpu.sync_copy(rows_hbm.at[pl.ds(base, b_per_w)], rows_v)
        pltpu.async_copy(rows_v, out_ref.at[idx_v], sem).wait()

    return scatter_k


def _gather_rows(table, idx_flat):
    V, D = table.shape
    return _make_sc_gather(V, D, idx_flat.shape[0])(table, idx_flat)


def _scatter_rows(out_ref, rows, idx_flat):
    _make_sc_scatter(rows.shape[1], rows.shape[0])(rows, idx_flat, out_ref)


def _rms(h, g, eps=1e-6):
    norm = lax.rsqrt(jnp.mean(h * h, axis=-1, keepdims=True) + eps)
    return h * norm * g


def _attn_body(x_ref, g1_ref, wq_ref, wk_ref, wv_ref, o_ref, h1_ref):
    hpair = pl.program_id(1)

    @pl.when(hpair == 0)
    def _():
        h1_ref[...] = _rms(x_ref[...], g1_ref[...])

    h1 = h1_ref[...]
    hp = lax.Precision.DEFAULT
    q2 = lax.dot_general(h1, wq_ref[...], (((1,), (0,)), ((), ())),
                         precision=hp, preferred_element_type=jnp.float32)
    k2 = lax.dot_general(h1, wk_ref[...], (((1,), (0,)), ((), ())),
                         precision=hp, preferred_element_type=jnp.float32)
    v2 = lax.dot_general(h1, wv_ref[...], (((1,), (0,)), ((), ())),
                         precision=hp, preferred_element_type=jnp.float32)
    Tn = q2.shape[0]
    QT = 256
    scale = 1.0 / math.sqrt(64.0)
    outs = []
    for j in range(2):
        q = q2[:, j * 64:(j + 1) * 64]
        k = k2[:, j * 64:(j + 1) * 64]
        v = v2[:, j * 64:(j + 1) * 64]
        otiles = []
        for qt in range(Tn // QT):
            ext = (qt + 1) * QT
            qtile = q[qt * QT:ext, :]
            s = lax.dot_general(qtile, k[:ext, :], (((1,), (1,)), ((), ())),
                                precision=hp,
                                preferred_element_type=jnp.float32) * scale
            rio = lax.broadcasted_iota(jnp.int32, (QT, ext), 0) + qt * QT
            cio = lax.broadcasted_iota(jnp.int32, (QT, ext), 1)
            s = jnp.where(cio > rio, -1e30, s)
            mx = jnp.max(s, axis=1, keepdims=True)
            e = jnp.exp(s - mx)
            pmat = e / jnp.sum(e, axis=1, keepdims=True)
            otiles.append(lax.dot_general(pmat, v[:ext, :],
                                          (((1,), (0,)), ((), ())),
                                          precision=hp,
                                          preferred_element_type=jnp.float32))
        outs.append(jnp.concatenate(otiles, axis=0))
    o_ref[...] = jnp.concatenate(outs, axis=1)


def _attention(xs, g1, Wqkv, B, D):
    N = xs.shape[0]
    Tn = N // B
    HP = N_HEADS // 2
    return pl.pallas_call(
        _attn_body,
        grid=(B, HP),
        in_specs=[
            pl.BlockSpec((Tn, D), lambda b, h: (b, 0)),
            pl.BlockSpec((1, D), lambda b, h: (0, 0)),
            pl.BlockSpec((D, 128), lambda b, h: (0, h)),
            pl.BlockSpec((D, 128), lambda b, h: (0, h + HP)),
            pl.BlockSpec((D, 128), lambda b, h: (0, h + 2 * HP)),
        ],
        out_specs=pl.BlockSpec((Tn, 128), lambda b, h: (b, h)),
        out_shape=jax.ShapeDtypeStruct((N, D), jnp.float32),
        scratch_shapes=[pltpu.VMEM((Tn, D), jnp.float32)],
    )(xs, g1, Wqkv, Wqkv, Wqkv)


def _ffn_body(x_ref, a_ref, g2_ref, wo_ref, w1_ref, w2_ref, w3_ref, o_ref):
    xs = x_ref[...]
    y = xs + lax.dot_general(a_ref[...], wo_ref[...],
                             (((1,), (0,)), ((), ())),
                             preferred_element_type=jnp.float32)
    h2 = _rms(y, g2_ref[...])
    a = lax.dot_general(h2, w1_ref[...], (((1,), (0,)), ((), ())),
                        preferred_element_type=jnp.float32)
    b = lax.dot_general(h2, w2_ref[...], (((1,), (0,)), ((), ())),
                        preferred_element_type=jnp.float32)
    act = (a / (1.0 + jnp.exp(-a))) * b
    ff = lax.dot_general(act, w3_ref[...], (((1,), (0,)), ((), ())),
                         preferred_element_type=jnp.float32)
    o_ref[...] = y + ff


def _out_ffn(xs, attn, g2, Wo, W1, W2, W3):
    N, D = xs.shape
    F = W1.shape[1]
    TM = 512
    return pl.pallas_call(
        _ffn_body,
        grid=(N // TM,),
        in_specs=[
            pl.BlockSpec((TM, D), lambda i: (i, 0)),
            pl.BlockSpec((TM, D), lambda i: (i, 0)),
            pl.BlockSpec((1, D), lambda i: (0, 0)),
            pl.BlockSpec((D, D), lambda i: (0, 0)),
            pl.BlockSpec((D, F), lambda i: (0, 0)),
            pl.BlockSpec((D, F), lambda i: (0, 0)),
            pl.BlockSpec((F, D), lambda i: (0, 0)),
        ],
        out_specs=pl.BlockSpec((TM, D), lambda i: (i, 0)),
        out_shape=jax.ShapeDtypeStruct((N, D), jnp.float32),
    )(xs, attn, g2, Wo, W1, W2, W3)


def kernel(x, Wr, g1, g2, Wqkv, Wo, W1, W2, W3):
    B, T, D = x.shape
    x2d = x.reshape(B * T, D)
    out = _make_sc_copy(B * T, D)(x2d)
    scores = _router_scores(x, Wr)
    idx_g = _topk_global_idx(scores)
    idx_flat = idx_g.reshape(B * CAPACITY)
    xs = _gather_rows(x2d, idx_flat)
    attn = _attention(xs, g1.reshape(1, D), Wqkv, B, D)
    xproc = _out_ffn(xs, attn, g2.reshape(1, D), Wo, W1, W2, W3)
    out_ref = jax.new_ref(out)
    _scatter_rows(out_ref, xproc, idx_flat)
    return jax.freeze(out_ref).reshape(B, T, D)


# speedup vs baseline: 10.0414x; 1.1208x over previous
import functools
import math

import jax
import jax.numpy as jnp
from jax import lax
from jax.experimental import pallas as pl
from jax.experimental.pallas import tpu as pltpu
from jax.experimental.pallas import tpu_sc as plsc

N_HEADS = 12
CAPACITY = 1024
_INT_MIN_PY = -2147483648


def _scores_body(x_ref, wr_ref, s_ref):
    xb = x_ref[...]
    Bb, TB, D = xb.shape
    s = lax.dot_general(xb.reshape(Bb * TB, D), wr_ref[...],
                        (((1,), (0,)), ((), ())),
                        precision=lax.Precision.HIGHEST,
                        preferred_element_type=jnp.float32)
    s_ref[...] = s.reshape(Bb, TB)


def _router_scores(x, Wr):
    B, T, D = x.shape
    TB = 512
    return pl.pallas_call(
        _scores_body,
        grid=(T // TB,),
        in_specs=[
            pl.BlockSpec((B, TB, D), lambda i: (0, i, 0)),
            pl.BlockSpec((D, 1), lambda i: (0, 0)),
        ],
        out_specs=pl.BlockSpec((B, TB), lambda i: (0, i)),
        out_shape=jax.ShapeDtypeStruct((B, T), jnp.float32),
    )(x, Wr)


def _make_sc_copy(N, D):
    mesh, nc, nw = _sc_mesh_info()
    rows_per_w = N // nw
    CHUNK = 64
    nch = rows_per_w // CHUNK

    @functools.partial(
        pl.kernel, mesh=mesh,
        out_type=jax.ShapeDtypeStruct((N, D), jnp.float32),
        cost_estimate=pl.CostEstimate(
            flops=0, transcendentals=0,
            bytes_accessed=2 * N * D * 4),
        scratch_types=[
            pltpu.VMEM((CHUNK, D), jnp.float32),
            pltpu.VMEM((CHUNK, D), jnp.float32),
            pltpu.SemaphoreType.DMA,
            pltpu.SemaphoreType.DMA,
            pltpu.SemaphoreType.DMA,
            pltpu.SemaphoreType.DMA,
        ],
    )
    def copy_k(x_hbm, out_hbm, buf0, buf1, in0, in1, out0, out1):
        wid = lax.axis_index("s") * nc + lax.axis_index("c")
        base = wid * rows_per_w
        bufs = (buf0, buf1)
        insems = (in0, in1)
        outsems = (out0, out1)
        for c in range(nch):
            sl = c % 2
            r0 = base + c * CHUNK
            if c >= 2:
                pltpu.make_async_copy(bufs[sl], out_hbm.at[pl.ds(r0, CHUNK)],
                                      outsems[sl]).wait()
            cp_in = pltpu.make_async_copy(x_hbm.at[pl.ds(r0, CHUNK)],
                                          bufs[sl], insems[sl])
            cp_in.start()
            cp_in.wait()
            pltpu.make_async_copy(bufs[sl], out_hbm.at[pl.ds(r0, CHUNK)],
                                  outsems[sl]).start()
        for sl in range(2):
            c = nch - 2 + sl
            r0 = base + c * CHUNK
            pltpu.make_async_copy(bufs[c % 2], out_hbm.at[pl.ds(r0, CHUNK)],
                                  outsems[c % 2]).wait()

    return copy_k


def _roll(x, d, axis):
    d = d % x.shape[axis]
    if d == 0:
        return x
    if axis == 0:
        return jnp.concatenate([x[d:, :], x[:d, :]], axis=0)
    return jnp.concatenate([x[:, d:], x[:, :d]], axis=1)


def _topk_body(s_ref, idx_ref):
    s = s_ref[...]
    B, T = s.shape
    K = CAPACITY
    INT_MIN = jnp.int32(_INT_MIN_PY)
    bits = lax.bitcast_convert_type(s, jnp.int32)
    key = jnp.where(bits >= 0, bits, INT_MIN - bits)

    R, C = T // 128, 128
    LOG = (T - 1).bit_length()
    rows_io = lax.broadcasted_iota(jnp.int32, (R, C), 0)
    cols_io = lax.broadcasted_iota(jnp.int32, (R, C), 1)
    lin = rows_io * C + cols_io
    up_masks = [(lin & (1 << j)) == 0 for j in range(LOG)]
    desc_masks = [((lin >> kk) & 1) == 0 for kk in range(1, LOG)]

    out_rows = []
    for b in range(B):
        k_arr = key[b].reshape(R, C)
        g_arr = lin
        for kk in range(1, LOG + 1):
            for j in range(kk - 1, -1, -1):
                d = 1 << j
                up = up_masks[j]
                if d < C:
                    pk = jnp.where(up, _roll(k_arr, d, 1), _roll(k_arr, -d, 1))
                    pg = jnp.where(up, _roll(g_arr, d, 1), _roll(g_arr, -d, 1))
                else:
                    m = d // C
                    pk = jnp.where(up, _roll(k_arr, m, 0), _roll(k_arr, -m, 0))
                    pg = jnp.where(up, _roll(g_arr, m, 0), _roll(g_arr, -m, 0))
                beats = (k_arr > pk) | ((k_arr == pk) & (g_arr < pg))
                if kk == LOG:
                    keep = beats == up
                else:
                    keep = beats == (up == desc_masks[kk - 1])
                k_arr = jnp.where(keep, k_arr, pk)
                g_arr = jnp.where(keep, g_arr, pg)
        top = g_arr[:K // C, :].reshape(1, K)
        out_rows.append(top + b * T)
    idx_ref[...] = jnp.concatenate(out_rows, axis=0)


def _topk_global_idx(scores):
    B, T = scores.shape
    return pl.pallas_call(
        _topk_body,
        out_shape=jax.ShapeDtypeStruct((B, CAPACITY), jnp.int32),
    )(scores)


def _sc_mesh_info():
    info = plsc.get_sparse_core_info()
    return (plsc.VectorSubcoreMesh(core_axis_name="c", subcore_axis_name="s"),
            info.num_cores, info.num_cores * info.num_subcores)


def _make_sc_gather(V, D, Bn):
    mesh, nc, nw = _sc_mesh_info()
    b_per_w = Bn // nw

    @functools.partial(
        pl.kernel, mesh=mesh,
        out_type=jax.ShapeDtypeStruct((Bn, D), jnp.float32),
        scratch_types=[
            pltpu.VMEM((b_per_w,), jnp.int32),
            pltpu.VMEM((b_per_w, D), jnp.float32),
            pltpu.SemaphoreType.DMA,
        ],
    )
    def gather_k(table_hbm, idx_hbm, out_hbm, idx_v, rows_v, sem):
        wid = lax.axis_index("s") * nc + lax.axis_index("c")
        base = wid * b_per_w
        pltpu.sync_copy(idx_hbm.at[pl.ds(base, b_per_w)], idx_v)
        pltpu.async_copy(table_hbm.at[idx_v], rows_v, sem).wait()
        pltpu.sync_copy(rows_v, out_hbm.at[pl.ds(base, b_per_w)])

    return gather_k


def _make_sc_scatter(D, Bn):
    mesh, nc, nw = _sc_mesh_info()
    b_per_w = Bn // nw

    @functools.partial(
        pl.kernel, mesh=mesh,
        out_type=(),
        scratch_types=[
            pltpu.VMEM((b_per_w,), jnp.int32),
            pltpu.VMEM((b_per_w, D), jnp.float32),
            pltpu.SemaphoreType.DMA,
        ],
    )
    def scatter_k(rows_hbm, idx_hbm, out_ref, idx_v, rows_v, sem):
        wid = lax.axis_index("s") * nc + lax.axis_index("c")
        base = wid * b_per_w
        pltpu.sync_copy(idx_hbm.at[pl.ds(base, b_per_w)], idx_v)
        pltpu.sync_copy(rows_hbm.at[pl.ds(base, b_per_w)], rows_v)
        pltpu.async_copy(rows_v, out_ref.at[idx_v], sem).wait()

    return scatter_k


def _gather_rows(table, idx_flat):
    V, D = table.shape
    return _make_sc_gather(V, D, idx_flat.shape[0])(table, idx_flat)


def _scatter_rows(out_ref, rows, idx_flat):
    _make_sc_scatter(rows.shape[1], rows.shape[0])(rows, idx_flat, out_ref)


def _rms(h, g, eps=1e-6):
    norm = lax.rsqrt(jnp.mean(h * h, axis=-1, keepdims=True) + eps)
    return h * norm * g


def _attn_body(x_ref, g1_ref, wq_ref, wk_ref, wv_ref, o_ref, h1_ref):
    hpair = pl.program_id(1)

    @pl.when(hpair == 0)
    def _():
        h1_ref[...] = _rms(x_ref[...], g1_ref[...])

    h1 = h1_ref[...]
    hp = lax.Precision.DEFAULT
    q2 = lax.dot_general(h1, wq_ref[...], (((1,), (0,)), ((), ())),
                         precision=hp, preferred_element_type=jnp.float32)
    k2 = lax.dot_general(h1, wk_ref[...], (((1,), (0,)), ((), ())),
                         precision=hp, preferred_element_type=jnp.float32)
    v2 = lax.dot_general(h1, wv_ref[...], (((1,), (0,)), ((), ())),
                         precision=hp, preferred_element_type=jnp.float32)
    Tn = q2.shape[0]
    QT = 256
    scale = 1.0 / math.sqrt(64.0)
    outs = []
    for j in range(2):
        q = q2[:, j * 64:(j + 1) * 64]
        k = k2[:, j * 64:(j + 1) * 64]
        v = v2[:, j * 64:(j + 1) * 64]
        otiles = []
        rio = lax.broadcasted_iota(jnp.int32, (QT, QT), 0)
        cio = lax.broadcasted_iota(jnp.int32, (QT, QT), 1)
        for qt in range(Tn // QT):
            ext = (qt + 1) * QT
            qtile = q[qt * QT:ext, :]
            s = lax.dot_general(qtile, k[:ext, :], (((1,), (1,)), ((), ())),
                                precision=hp,
                                preferred_element_type=jnp.float32) * scale
            e = jnp.exp(s)
            ed = jnp.where(cio > rio, 0.0, e[:, qt * QT:ext])
            if qt > 0:
                e = jnp.concatenate([e[:, :qt * QT], ed], axis=1)
            else:
                e = ed
            denom = jnp.sum(e, axis=1, keepdims=True)
            ov = lax.dot_general(e, v[:ext, :], (((1,), (0,)), ((), ())),
                                 precision=hp,
                                 preferred_element_type=jnp.float32)
            otiles.append(ov / denom)
        outs.append(jnp.concatenate(otiles, axis=0))
    o_ref[...] = jnp.concatenate(outs, axis=1)


def _attention(xs, g1, Wqkv, B, D):
    N = xs.shape[0]
    Tn = N // B
    HP = N_HEADS // 2
    return pl.pallas_call(
        _attn_body,
        grid=(B, HP),
        in_specs=[
            pl.BlockSpec((Tn, D), lambda b, h: (b, 0)),
            pl.BlockSpec((1, D), lambda b, h: (0, 0)),
            pl.BlockSpec((D, 128), lambda b, h: (0, h)),
            pl.BlockSpec((D, 128), lambda b, h: (0, h + HP)),
            pl.BlockSpec((D, 128), lambda b, h: (0, h + 2 * HP)),
        ],
        out_specs=pl.BlockSpec((Tn, 128), lambda b, h: (b, h)),
        out_shape=jax.ShapeDtypeStruct((N, D), jnp.float32),
        scratch_shapes=[pltpu.VMEM((Tn, D), jnp.float32)],
    )(xs, g1, Wqkv, Wqkv, Wqkv)


def _ffn_body(x_ref, a_ref, g2_ref, wo_ref, w1_ref, w2_ref, w3_ref, o_ref):
    xs = x_ref[...]
    y = xs + lax.dot_general(a_ref[...], wo_ref[...],
                             (((1,), (0,)), ((), ())),
                             preferred_element_type=jnp.float32)
    h2 = _rms(y, g2_ref[...])
    a = lax.dot_general(h2, w1_ref[...], (((1,), (0,)), ((), ())),
                        preferred_element_type=jnp.float32)
    b = lax.dot_general(h2, w2_ref[...], (((1,), (0,)), ((), ())),
                        preferred_element_type=jnp.float32)
    act = (a / (1.0 + jnp.exp(-a))) * b
    ff = lax.dot_general(act, w3_ref[...], (((1,), (0,)), ((), ())),
                         preferred_element_type=jnp.float32)
    o_ref[...] = y + ff


def _out_ffn(xs, attn, g2, Wo, W1, W2, W3):
    N, D = xs.shape
    F = W1.shape[1]
    TM = 512
    return pl.pallas_call(
        _ffn_body,
        grid=(N // TM,),
        in_specs=[
            pl.BlockSpec((TM, D), lambda i: (i, 0)),
            pl.BlockSpec((TM, D), lambda i: (i, 0)),
            pl.BlockSpec((1, D), lambda i: (0, 0)),
            pl.BlockSpec((D, D), lambda i: (0, 0)),
            pl.BlockSpec((D, F), lambda i: (0, 0)),
            pl.BlockSpec((D, F), lambda i: (0, 0)),
            pl.BlockSpec((F, D), lambda i: (0, 0)),
        ],
        out_specs=pl.BlockSpec((TM, D), lambda i: (i, 0)),
        out_shape=jax.ShapeDtypeStruct((N, D), jnp.float32),
    )(xs, attn, g2, Wo, W1, W2, W3)


def kernel(x, Wr, g1, g2, Wqkv, Wo, W1, W2, W3):
    B, T, D = x.shape
    x2d = x.reshape(B * T, D)
    out = _make_sc_copy(B * T, D)(x2d)
    scores = _router_scores(x, Wr)
    idx_g = _topk_global_idx(scores)
    idx_flat = idx_g.reshape(B * CAPACITY)
    xs = _gather_rows(x2d, idx_flat)
    attn = _attention(xs, g1.reshape(1, D), Wqkv, B, D)
    xproc = _out_ffn(xs, attn, g2.reshape(1, D), Wo, W1, W2, W3)
    out_ref = jax.new_ref(out)
    _scatter_rows(out_ref, xproc, idx_flat)
    return jax.freeze(out_ref).reshape(B, T, D)


# speedup vs baseline: 10.1178x; 1.0076x over previous
import functools
import math

import jax
import jax.numpy as jnp
from jax import lax
from jax.experimental import pallas as pl
from jax.experimental.pallas import tpu as pltpu
from jax.experimental.pallas import tpu_sc as plsc

N_HEADS = 12
CAPACITY = 1024
_INT_MIN_PY = -2147483648


def _scores_copy_body(x_ref, wr_ref, out_ref, s_ref):
    xb = x_ref[...]
    out_ref[...] = xb
    Bb, TB, D = xb.shape
    s = lax.dot_general(xb.reshape(Bb * TB, D), wr_ref[...],
                        (((1,), (0,)), ((), ())),
                        precision=lax.Precision.HIGHEST,
                        preferred_element_type=jnp.float32)
    s_ref[...] = s.reshape(Bb, TB)


def _scores_and_copy(x, Wr):
    B, T, D = x.shape
    TB = 512
    out, scores = pl.pallas_call(
        _scores_copy_body,
        grid=(T // TB,),
        in_specs=[
            pl.BlockSpec((B, TB, D), lambda i: (0, i, 0)),
            pl.BlockSpec((D, 1), lambda i: (0, 0)),
        ],
        out_specs=[
            pl.BlockSpec((B, TB, D), lambda i: (0, i, 0)),
            pl.BlockSpec((B, TB), lambda i: (0, i)),
        ],
        out_shape=[
            jax.ShapeDtypeStruct((B, T, D), jnp.float32),
            jax.ShapeDtypeStruct((B, T), jnp.float32),
        ],
    )(x, Wr)
    return out, scores


def _roll(x, d, axis):
    d = d % x.shape[axis]
    if d == 0:
        return x
    if axis == 0:
        return jnp.concatenate([x[d:, :], x[:d, :]], axis=0)
    return jnp.concatenate([x[:, d:], x[:, :d]], axis=1)


def _topk_body(s_ref, idx_ref):
    s = s_ref[...]
    B, T = s.shape
    K = CAPACITY
    INT_MIN = jnp.int32(_INT_MIN_PY)
    bits = lax.bitcast_convert_type(s, jnp.int32)
    key = jnp.where(bits >= 0, bits, INT_MIN - bits)

    R, C = T // 128, 128
    LOG = (T - 1).bit_length()
    rows_io = lax.broadcasted_iota(jnp.int32, (R, C), 0)
    cols_io = lax.broadcasted_iota(jnp.int32, (R, C), 1)
    lin = rows_io * C + cols_io
    up_masks = [(lin & (1 << j)) == 0 for j in range(LOG)]
    desc_masks = [((lin >> kk) & 1) == 0 for kk in range(1, LOG)]

    out_rows = []
    for b in range(B):
        k_arr = key[b].reshape(R, C)
        g_arr = lin
        for kk in range(1, LOG + 1):
            for j in range(kk - 1, -1, -1):
                d = 1 << j
                up = up_masks[j]
                if d < C:
                    pk = jnp.where(up, _roll(k_arr, d, 1), _roll(k_arr, -d, 1))
                    pg = jnp.where(up, _roll(g_arr, d, 1), _roll(g_arr, -d, 1))
                else:
                    m = d // C
                    pk = jnp.where(up, _roll(k_arr, m, 0), _roll(k_arr, -m, 0))
                    pg = jnp.where(up, _roll(g_arr, m, 0), _roll(g_arr, -m, 0))
                beats = (k_arr > pk) | ((k_arr == pk) & (g_arr < pg))
                if kk == LOG:
                    keep = beats == up
                else:
                    keep = beats == (up == desc_masks[kk - 1])
                k_arr = jnp.where(keep, k_arr, pk)
                g_arr = jnp.where(keep, g_arr, pg)
        top = g_arr[:K // C, :].reshape(1, K)
        out_rows.append(top + b * T)
    idx_ref[...] = jnp.concatenate(out_rows, axis=0)


def _topk_global_idx(scores):
    B, T = scores.shape
    return pl.pallas_call(
        _topk_body,
        out_shape=jax.ShapeDtypeStruct((B, CAPACITY), jnp.int32),
    )(scores)


def _sc_mesh_info():
    info = plsc.get_sparse_core_info()
    return (plsc.VectorSubcoreMesh(core_axis_name="c", subcore_axis_name="s"),
            info.num_cores, info.num_cores * info.num_subcores)


def _make_sc_gather(V, D, Bn):
    mesh, nc, nw = _sc_mesh_info()
    b_per_w = Bn // nw

    @functools.partial(
        pl.kernel, mesh=mesh,
        out_type=jax.ShapeDtypeStruct((Bn, D), jnp.float32),
        scratch_types=[
            pltpu.VMEM((b_per_w,), jnp.int32),
            pltpu.VMEM((b_per_w, D), jnp.float32),
            pltpu.SemaphoreType.DMA,
        ],
    )
    def gather_k(table_hbm, idx_hbm, out_hbm, idx_v, rows_v, sem):
        wid = lax.axis_index("s") * nc + lax.axis_index("c")
        base = wid * b_per_w
        pltpu.sync_copy(idx_hbm.at[pl.ds(base, b_per_w)], idx_v)
        pltpu.async_copy(table_hbm.at[idx_v], rows_v, sem).wait()
        pltpu.sync_copy(rows_v, out_hbm.at[pl.ds(base, b_per_w)])

    return gather_k


def _make_sc_scatter(D, Bn):
    mesh, nc, nw = _sc_mesh_info()
    b_per_w = Bn // nw

    @functools.partial(
        pl.kernel, mesh=mesh,
        out_type=(),
        scratch_types=[
            pltpu.VMEM((b_per_w,), jnp.int32),
            pltpu.VMEM((b_per_w, D), jnp.float32),
            pltpu.SemaphoreType.DMA,
        ],
    )
    def scatter_k(rows_hbm, idx_hbm, out_ref, idx_v, rows_v, sem):
        wid = lax.axis_index("s") * nc + lax.axis_index("c")
        base = wid * b_per_w
        pltpu.sync_copy(idx_hbm.at[pl.ds(base, b_per_w)], idx_v)
        pltpu.sync_copy(rows_hbm.at[pl.ds(base, b_per_w)], rows_v)
        pltpu.async_copy(rows_v, out_ref.at[idx_v], sem).wait()

    return scatter_k


def _gather_rows(table, idx_flat):
    V, D = table.shape
    return _make_sc_gather(V, D, idx_flat.shape[0])(table, idx_flat)


def _scatter_rows(out_ref, rows, idx_flat):
    _make_sc_scatter(rows.shape[1], rows.shape[0])(rows, idx_flat, out_ref)


def _rms(h, g, eps=1e-6):
    norm = lax.rsqrt(jnp.mean(h * h, axis=-1, keepdims=True) + eps)
    return h * norm * g


def _attn_body(x_ref, g1_ref, wq_ref, wk_ref, wv_ref, o_ref, h1_ref):
    hpair = pl.program_id(1)

    @pl.when(hpair == 0)
    def _():
        h1_ref[...] = _rms(x_ref[...], g1_ref[...])

    h1 = h1_ref[...]
    hp = lax.Precision.DEFAULT
    q2 = lax.dot_general(h1, wq_ref[...], (((1,), (0,)), ((), ())),
                         precision=hp, preferred_element_type=jnp.float32)
    k2 = lax.dot_general(h1, wk_ref[...], (((1,), (0,)), ((), ())),
                         precision=hp, preferred_element_type=jnp.float32)
    v2 = lax.dot_general(h1, wv_ref[...], (((1,), (0,)), ((), ())),
                         precision=hp, preferred_element_type=jnp.float32)
    Tn = q2.shape[0]
    QT = 256
    scale = 1.0 / math.sqrt(64.0)
    outs = []
    for j in range(2):
        q = q2[:, j * 64:(j + 1) * 64]
        k = k2[:, j * 64:(j + 1) * 64]
        v = v2[:, j * 64:(j + 1) * 64]
        otiles = []
        rio = lax.broadcasted_iota(jnp.int32, (QT, QT), 0)
        cio = lax.broadcasted_iota(jnp.int32, (QT, QT), 1)
        for qt in range(Tn // QT):
            ext = (qt + 1) * QT
            qtile = q[qt * QT:ext, :]
            s = lax.dot_general(qtile, k[:ext, :], (((1,), (1,)), ((), ())),
                                precision=hp,
                                preferred_element_type=jnp.float32) * scale
            e = jnp.exp(s)
            ed = jnp.where(cio > rio, 0.0, e[:, qt * QT:ext])
            if qt > 0:
                e = jnp.concatenate([e[:, :qt * QT], ed], axis=1)
            else:
                e = ed
            denom = jnp.sum(e, axis=1, keepdims=True)
            ov = lax.dot_general(e, v[:ext, :], (((1,), (0,)), ((), ())),
                                 precision=hp,
                                 preferred_element_type=jnp.float32)
            otiles.append(ov / denom)
        outs.append(jnp.concatenate(otiles, axis=0))
    o_ref[...] = jnp.concatenate(outs, axis=1)


def _attention(xs, g1, Wqkv, B, D):
    N = xs.shape[0]
    Tn = N // B
    HP = N_HEADS // 2
    return pl.pallas_call(
        _attn_body,
        grid=(B, HP),
        in_specs=[
            pl.BlockSpec((Tn, D), lambda b, h: (b, 0)),
            pl.BlockSpec((1, D), lambda b, h: (0, 0)),
            pl.BlockSpec((D, 128), lambda b, h: (0, h)),
            pl.BlockSpec((D, 128), lambda b, h: (0, h + HP)),
            pl.BlockSpec((D, 128), lambda b, h: (0, h + 2 * HP)),
        ],
        out_specs=pl.BlockSpec((Tn, 128), lambda b, h: (b, h)),
        out_shape=jax.ShapeDtypeStruct((N, D), jnp.float32),
        scratch_shapes=[pltpu.VMEM((Tn, D), jnp.float32)],
    )(xs, g1, Wqkv, Wqkv, Wqkv)


def _ffn_body(x_ref, a_ref, g2_ref, wo_ref, w1_ref, w2_ref, w3_ref, o_ref):
    xs = x_ref[...]
    y = xs + lax.dot_general(a_ref[...], wo_ref[...],
                             (((1,), (0,)), ((), ())),
                             preferred_element_type=jnp.float32)
    h2 = _rms(y, g2_ref[...])
    a = lax.dot_general(h2, w1_ref[...], (((1,), (0,)), ((), ())),
                        preferred_element_type=jnp.float32)
    b = lax.dot_general(h2, w2_ref[...], (((1,), (0,)), ((), ())),
                        preferred_element_type=jnp.float32)
    act = (a / (1.0 + jnp.exp(-a))) * b
    ff = lax.dot_general(act, w3_ref[...], (((1,), (0,)), ((), ())),
                         preferred_element_type=jnp.float32)
    o_ref[...] = y + ff


def _out_ffn(xs, attn, g2, Wo, W1, W2, W3):
    N, D = xs.shape
    F = W1.shape[1]
    TM = 512
    return pl.pallas_call(
        _ffn_body,
        grid=(N // TM,),
        in_specs=[
            pl.BlockSpec((TM, D), lambda i: (i, 0)),
            pl.BlockSpec((TM, D), lambda i: (i, 0)),
            pl.BlockSpec((1, D), lambda i: (0, 0)),
            pl.BlockSpec((D, D), lambda i: (0, 0)),
            pl.BlockSpec((D, F), lambda i: (0, 0)),
            pl.BlockSpec((D, F), lambda i: (0, 0)),
            pl.BlockSpec((F, D), lambda i: (0, 0)),
        ],
        out_specs=pl.BlockSpec((TM, D), lambda i: (i, 0)),
        out_shape=jax.ShapeDtypeStruct((N, D), jnp.float32),
    )(xs, attn, g2, Wo, W1, W2, W3)


def kernel(x, Wr, g1, g2, Wqkv, Wo, W1, W2, W3):
    B, T, D = x.shape
    x2d = x.reshape(B * T, D)
    out, scores = _scores_and_copy(x, Wr)
    idx_g = _topk_global_idx(scores)
    idx_flat = idx_g.reshape(B * CAPACITY)
    xs = _gather_rows(x2d, idx_flat)
    attn = _attention(xs, g1.reshape(1, D), Wqkv, B, D)
    xproc = _out_ffn(xs, attn, g2.reshape(1, D), Wo, W1, W2, W3)
    out_ref = jax.new_ref(out.reshape(B * T, D))
    _scatter_rows(out_ref, xproc, idx_flat)
    return jax.freeze(out_ref).reshape(B, T, D)


# speedup vs baseline: 10.2137x; 1.0095x over previous
import functools
import math

import jax
import jax.numpy as jnp
from jax import lax
from jax.experimental import pallas as pl
from jax.experimental.pallas import tpu as pltpu
from jax.experimental.pallas import tpu_sc as plsc

N_HEADS = 12
CAPACITY = 1024
_INT_MIN_PY = -2147483648


def _scores_copy_body(x_ref, wr_ref, out_ref, s_ref):
    xb = x_ref[...]
    out_ref[...] = xb
    Bb, TB, D = xb.shape
    s = lax.dot_general(xb.reshape(Bb * TB, D), wr_ref[...],
                        (((1,), (0,)), ((), ())),
                        precision=lax.Precision.HIGHEST,
                        preferred_element_type=jnp.float32)
    s_ref[...] = s.reshape(Bb, TB)


def _scores_and_copy(x, Wr):
    B, T, D = x.shape
    TB = 512
    out, scores = pl.pallas_call(
        _scores_copy_body,
        grid=(T // TB,),
        in_specs=[
            pl.BlockSpec((B, TB, D), lambda i: (0, i, 0)),
            pl.BlockSpec((D, 1), lambda i: (0, 0)),
        ],
        out_specs=[
            pl.BlockSpec((B, TB, D), lambda i: (0, i, 0)),
            pl.BlockSpec((B, TB), lambda i: (0, i)),
        ],
        out_shape=[
            jax.ShapeDtypeStruct((B, T, D), jnp.float32),
            jax.ShapeDtypeStruct((B, T), jnp.float32),
        ],
    )(x, Wr)
    return out, scores


def _roll(x, d, axis):
    d = d % x.shape[axis]
    if d == 0:
        return x
    if axis == 0:
        return jnp.concatenate([x[d:, :], x[:d, :]], axis=0)
    return jnp.concatenate([x[:, d:], x[:, :d]], axis=1)


def _topk_body(s_ref, idx_ref):
    s = s_ref[...]
    B, T = s.shape
    K = CAPACITY
    INT_MIN = jnp.int32(_INT_MIN_PY)
    bits = lax.bitcast_convert_type(s, jnp.int32)
    key = jnp.where(bits >= 0, bits, INT_MIN - bits)

    R, C = T // 128, 128
    LOG = (T - 1).bit_length()
    rows_io = lax.broadcasted_iota(jnp.int32, (R, C), 0)
    cols_io = lax.broadcasted_iota(jnp.int32, (R, C), 1)
    lin = rows_io * C + cols_io
    up_masks = [(lin & (1 << j)) == 0 for j in range(LOG)]
    desc_masks = [((lin >> kk) & 1) == 0 for kk in range(1, LOG)]

    out_rows = []
    for b in range(B):
        k_arr = key[b].reshape(R, C)
        g_arr = lin
        for kk in range(1, LOG + 1):
            for j in range(kk - 1, -1, -1):
                d = 1 << j
                up = up_masks[j]
                if d < C:
                    pk = jnp.where(up, _roll(k_arr, d, 1), _roll(k_arr, -d, 1))
                    pg = jnp.where(up, _roll(g_arr, d, 1), _roll(g_arr, -d, 1))
                else:
                    m = d // C
                    pk = jnp.where(up, _roll(k_arr, m, 0), _roll(k_arr, -m, 0))
                    pg = jnp.where(up, _roll(g_arr, m, 0), _roll(g_arr, -m, 0))
                beats = (k_arr > pk) | ((k_arr == pk) & (g_arr < pg))
                if kk == LOG:
                    keep = beats == up
                else:
                    keep = beats == (up == desc_masks[kk - 1])
                k_arr = jnp.where(keep, k_arr, pk)
                g_arr = jnp.where(keep, g_arr, pg)
        top = g_arr[:K // C, :].reshape(1, K)
        out_rows.append(top + b * T)
    idx_ref[...] = jnp.concatenate(out_rows, axis=0)


def _topk_global_idx(scores):
    B, T = scores.shape
    return pl.pallas_call(
        _topk_body,
        out_shape=jax.ShapeDtypeStruct((B, CAPACITY), jnp.int32),
    )(scores)


def _sc_mesh_info():
    info = plsc.get_sparse_core_info()
    return (plsc.VectorSubcoreMesh(core_axis_name="c", subcore_axis_name="s"),
            info.num_cores, info.num_cores * info.num_subcores)


def _make_sc_gather(V, D, Bn):
    mesh, nc, nw = _sc_mesh_info()
    b_per_w = Bn // nw

    @functools.partial(
        pl.kernel, mesh=mesh,
        out_type=jax.ShapeDtypeStruct((Bn, D), jnp.float32),
        scratch_types=[
            pltpu.VMEM((b_per_w,), jnp.int32),
            pltpu.VMEM((b_per_w, D), jnp.float32),
            pltpu.SemaphoreType.DMA,
        ],
    )
    def gather_k(table_hbm, idx_hbm, out_hbm, idx_v, rows_v, sem):
        wid = lax.axis_index("s") * nc + lax.axis_index("c")
        base = wid * b_per_w
        pltpu.sync_copy(idx_hbm.at[pl.ds(base, b_per_w)], idx_v)
        pltpu.async_copy(table_hbm.at[idx_v], rows_v, sem).wait()
        pltpu.sync_copy(rows_v, out_hbm.at[pl.ds(base, b_per_w)])

    return gather_k


def _make_sc_scatter(D, Bn):
    mesh, nc, nw = _sc_mesh_info()
    b_per_w = Bn // nw

    @functools.partial(
        pl.kernel, mesh=mesh,
        out_type=(),
        scratch_types=[
            pltpu.VMEM((b_per_w,), jnp.int32),
            pltpu.VMEM((b_per_w, D), jnp.float32),
            pltpu.SemaphoreType.DMA,
        ],
    )
    def scatter_k(rows_hbm, idx_hbm, out_ref, idx_v, rows_v, sem):
        wid = lax.axis_index("s") * nc + lax.axis_index("c")
        base = wid * b_per_w
        pltpu.sync_copy(idx_hbm.at[pl.ds(base, b_per_w)], idx_v)
        pltpu.sync_copy(rows_hbm.at[pl.ds(base, b_per_w)], rows_v)
        pltpu.async_copy(rows_v, out_ref.at[idx_v], sem).wait()

    return scatter_k


def _gather_rows(table, idx_flat):
    V, D = table.shape
    return _make_sc_gather(V, D, idx_flat.shape[0])(table, idx_flat)


def _scatter_rows(out_ref, rows, idx_flat):
    _make_sc_scatter(rows.shape[1], rows.shape[0])(rows, idx_flat, out_ref)


def _rms(h, g, eps=1e-6):
    norm = lax.rsqrt(jnp.mean(h * h, axis=-1, keepdims=True) + eps)
    return h * norm * g


def _attn_body(x_ref, g1_ref, wq_ref, wk_ref, wv_ref, o_ref, h1_ref):
    hpair = pl.program_id(1)

    @pl.when(hpair == 0)
    def _():
        h1_ref[...] = _rms(x_ref[...], g1_ref[...])

    h1 = h1_ref[...]
    hp = lax.Precision.DEFAULT
    q2 = lax.dot_general(h1, wq_ref[...], (((1,), (0,)), ((), ())),
                         precision=hp, preferred_element_type=jnp.float32)
    k2 = lax.dot_general(h1, wk_ref[...], (((1,), (0,)), ((), ())),
                         precision=hp, preferred_element_type=jnp.float32)
    v2 = lax.dot_general(h1, wv_ref[...], (((1,), (0,)), ((), ())),
                         precision=hp, preferred_element_type=jnp.float32)
    Tn = q2.shape[0]
    QT = 512
    scale = 1.0 / math.sqrt(64.0)
    outs = []
    for j in range(2):
        q = q2[:, j * 64:(j + 1) * 64]
        k = k2[:, j * 64:(j + 1) * 64]
        v = v2[:, j * 64:(j + 1) * 64]
        otiles = []
        rio = lax.broadcasted_iota(jnp.int32, (QT, QT), 0)
        cio = lax.broadcasted_iota(jnp.int32, (QT, QT), 1)
        for qt in range(Tn // QT):
            ext = (qt + 1) * QT
            qtile = q[qt * QT:ext, :]
            s = lax.dot_general(qtile, k[:ext, :], (((1,), (1,)), ((), ())),
                                precision=hp,
                                preferred_element_type=jnp.float32) * scale
            e = jnp.exp(s)
            ed = jnp.where(cio > rio, 0.0, e[:, qt * QT:ext])
            if qt > 0:
                e = jnp.concatenate([e[:, :qt * QT], ed], axis=1)
            else:
                e = ed
            denom = jnp.sum(e, axis=1, keepdims=True)
            ov = lax.dot_general(e, v[:ext, :], (((1,), (0,)), ((), ())),
                                 precision=hp,
                                 preferred_element_type=jnp.float32)
            otiles.append(ov / denom)
        outs.append(jnp.concatenate(otiles, axis=0))
    o_ref[...] = jnp.concatenate(outs, axis=1)


def _attention(xs, g1, Wqkv, B, D):
    N = xs.shape[0]
    Tn = N // B
    HP = N_HEADS // 2
    return pl.pallas_call(
        _attn_body,
        grid=(B, HP),
        in_specs=[
            pl.BlockSpec((Tn, D), lambda b, h: (b, 0)),
            pl.BlockSpec((1, D), lambda b, h: (0, 0)),
            pl.BlockSpec((D, 128), lambda b, h: (0, h)),
            pl.BlockSpec((D, 128), lambda b, h: (0, h + HP)),
            pl.BlockSpec((D, 128), lambda b, h: (0, h + 2 * HP)),
        ],
        out_specs=pl.BlockSpec((Tn, 128), lambda b, h: (b, h)),
        out_shape=jax.ShapeDtypeStruct((N, D), jnp.float32),
        scratch_shapes=[pltpu.VMEM((Tn, D), jnp.float32)],
    )(xs, g1, Wqkv, Wqkv, Wqkv)


def _ffn_body(x_ref, a_ref, g2_ref, wo_ref, w1_ref, w2_ref, w3_ref, o_ref):
    xs = x_ref[...]
    y = xs + lax.dot_general(a_ref[...], wo_ref[...],
                             (((1,), (0,)), ((), ())),
                             preferred_element_type=jnp.float32)
    h2 = _rms(y, g2_ref[...])
    a = lax.dot_general(h2, w1_ref[...], (((1,), (0,)), ((), ())),
                        preferred_element_type=jnp.float32)
    b = lax.dot_general(h2, w2_ref[...], (((1,), (0,)), ((), ())),
                        preferred_element_type=jnp.float32)
    act = (a / (1.0 + jnp.exp(-a))) * b
    ff = lax.dot_general(act, w3_ref[...], (((1,), (0,)), ((), ())),
                         preferred_element_type=jnp.float32)
    o_ref[...] = y + ff


def _out_ffn(xs, attn, g2, Wo, W1, W2, W3):
    N, D = xs.shape
    F = W1.shape[1]
    TM = 512
    return pl.pallas_call(
        _ffn_body,
        grid=(N // TM,),
        in_specs=[
            pl.BlockSpec((TM, D), lambda i: (i, 0)),
            pl.BlockSpec((TM, D), lambda i: (i, 0)),
            pl.BlockSpec((1, D), lambda i: (0, 0)),
            pl.BlockSpec((D, D), lambda i: (0, 0)),
            pl.BlockSpec((D, F), lambda i: (0, 0)),
            pl.BlockSpec((D, F), lambda i: (0, 0)),
            pl.BlockSpec((F, D), lambda i: (0, 0)),
        ],
        out_specs=pl.BlockSpec((TM, D), lambda i: (i, 0)),
        out_shape=jax.ShapeDtypeStruct((N, D), jnp.float32),
    )(xs, attn, g2, Wo, W1, W2, W3)


def kernel(x, Wr, g1, g2, Wqkv, Wo, W1, W2, W3):
    B, T, D = x.shape
    x2d = x.reshape(B * T, D)
    out, scores = _scores_and_copy(x, Wr)
    idx_g = _topk_global_idx(scores)
    idx_flat = idx_g.reshape(B * CAPACITY)
    xs = _gather_rows(x2d, idx_flat)
    attn = _attention(xs, g1.reshape(1, D), Wqkv, B, D)
    xproc = _out_ffn(xs, attn, g2.reshape(1, D), Wo, W1, W2, W3)
    out_ref = jax.new_ref(out.reshape(B * T, D))
    _scatter_rows(out_ref, xproc, idx_flat)
    return jax.freeze(out_ref).reshape(B, T, D)
